# Initial kernel scaffold; baseline (speedup 1.0000x reference)
import jax
import jax.numpy as jnp
from jax.experimental import pallas as pl


def kernel(seq1, mask1, seq2, mask2):
    raise NotImplementedError("write your pallas kernel here")


# speedup vs baseline: 2.4750x; 2.4750x over previous
import jax
import jax.numpy as jnp
from jax import lax
from jax.experimental import pallas as pl
from jax.experimental.pallas import tpu as pltpu
from jax.experimental.pallas import tpu_sc as plsc

_CHUNK = 16
_ZCHUNK = 32


def _meta_body(m1_ref, m2_ref, len_ref, mask_ref):
    m1 = m1_ref[...]
    m2 = m2_ref[...]
    len1 = m1.shape[1] - jnp.sum(m1, axis=1)
    len2 = m2.shape[1] - jnp.sum(m2, axis=1)
    total = len1 + len2
    len_ref[...] = jnp.broadcast_to(len1[:, None], len_ref.shape)
    r = lax.broadcasted_iota(jnp.int32, mask_ref.shape, 1)
    mask_ref[...] = (r >= total[:, None]).astype(jnp.int32)


def _sc_body(L1, L2, seq1_hbm, seq2_hbm, len_hbm, out_hbm, lenv, zbuf, buf):
    b = lax.axis_index("s")
    role = lax.axis_index("c")
    LT = L1 + L2
    H = zbuf.shape[1]

    pltpu.sync_copy(len_hbm.at[b], lenv)
    n1 = lenv[...][0]

    @pl.when(role == 0)
    def _():
        nfull = n1 // _CHUNK

        def chunk(i, carry):
            s0 = i * _CHUNK
            pltpu.sync_copy(seq1_hbm.at[pl.ds(s0, _CHUNK), b], buf)
            pltpu.sync_copy(buf, out_hbm.at[pl.ds(s0, _CHUNK), b])
            return carry

        lax.fori_loop(0, nfull, chunk, 0)

        def tail(i, carry):
            r0 = nfull * _CHUNK + i
            pltpu.sync_copy(seq1_hbm.at[r0, b], buf.at[0])
            pltpu.sync_copy(buf.at[0], out_hbm.at[r0, b])
            return carry

        lax.fori_loop(0, n1 - nfull * _CHUNK, tail, 0)

        def zinit(j, carry):
            row = j // (H // 16)
            col = (j % (H // 16)) * 16
            zbuf[row, pl.ds(col, 16)] = jnp.zeros((16,), jnp.float32)
            return carry

        lax.fori_loop(0, _ZCHUNK * (H // 16), zinit, 0)

        zstart = n1 + L2
        nz = L1 - n1
        nzfull = nz // _ZCHUNK

        def zchunk(i, carry):
            s0 = zstart + i * _ZCHUNK
            pltpu.sync_copy(zbuf, out_hbm.at[pl.ds(s0, _ZCHUNK), b])
            return carry

        lax.fori_loop(0, nzfull, zchunk, 0)

        def ztail(i, carry):
            r0 = zstart + nzfull * _ZCHUNK + i
            pltpu.sync_copy(zbuf.at[0], out_hbm.at[r0, b])
            return carry

        lax.fori_loop(0, nz - nzfull * _ZCHUNK, ztail, 0)

    @pl.when(role == 1)
    def _():
        def chunk(i, carry):
            s0 = i * _CHUNK
            pltpu.sync_copy(seq2_hbm.at[pl.ds(s0, _CHUNK), b], buf)
            pltpu.sync_copy(buf, out_hbm.at[pl.ds(n1 + s0, _CHUNK), b])
            return carry

        lax.fori_loop(0, L2 // _CHUNK, chunk, 0)


def kernel(seq1, mask1, seq2, mask2):
    L1, B, H = seq1.shape
    L2 = seq2.shape[0]
    LT = L1 + L2

    lens8, maski = pl.pallas_call(
        _meta_body,
        out_shape=(jax.ShapeDtypeStruct((B, 16), jnp.int32),
                   jax.ShapeDtypeStruct((B, LT), jnp.int32)),
    )(mask1.astype(jnp.int32), mask2.astype(jnp.int32))

    mesh = plsc.VectorSubcoreMesh(core_axis_name="c", subcore_axis_name="s")
    body = lambda *refs: _sc_body(L1, L2, *refs)
    out = pl.kernel(
        body,
        out_type=jax.ShapeDtypeStruct((LT, B, H), jnp.float32),
        mesh=mesh,
        scratch_types=[
            pltpu.VMEM((16,), jnp.int32),
            pltpu.VMEM((_ZCHUNK, H), jnp.float32),
            pltpu.VMEM((_CHUNK, H), jnp.float32),
        ],
    )(seq1, seq2, lens8)

    return (out, maski.astype(bool))


# speedup vs baseline: 3.7007x; 1.4952x over previous
import jax
import jax.numpy as jnp
from jax import lax
from jax.experimental import pallas as pl
from jax.experimental.pallas import tpu as pltpu
from jax.experimental.pallas import tpu_sc as plsc

_CHUNK = 32
_ZCHUNK = 32
_ZLAG = 8


def _meta_body(m1_ref, m2_ref, len_ref, mask_ref):
    m1 = m1_ref[...]
    m2 = m2_ref[...]
    len1 = m1.shape[1] - jnp.sum(m1, axis=1)
    len2 = m2.shape[1] - jnp.sum(m2, axis=1)
    total = len1 + len2
    len_ref[...] = jnp.broadcast_to(len1[:, None], len_ref.shape)
    r = lax.broadcasted_iota(jnp.int32, mask_ref.shape, 1)
    mask_ref[...] = (r >= total[:, None]).astype(jnp.int32)


def _copy_pipelined(src_hbm, dst_hbm, b, n_rows, dst_off, buf, isem, osem):
    C = buf.shape[1]
    nfull = n_rows // C

    def start_in(i, p):
        pltpu.async_copy(src_hbm.at[pl.ds(i * C, C), b], buf.at[p],
                         isem.at[p])

    def wait_in(i, p):
        pltpu.make_async_copy(src_hbm.at[pl.ds(i * C, C), b], buf.at[p],
                              isem.at[p]).wait()

    def start_out(i, p):
        pltpu.async_copy(buf.at[p], dst_hbm.at[pl.ds(dst_off + i * C, C), b],
                         osem.at[p])

    def wait_out(i, p):
        pltpu.make_async_copy(buf.at[p],
                              dst_hbm.at[pl.ds(dst_off + i * C, C), b],
                              osem.at[p]).wait()

    @pl.when(nfull > 0)
    def _():
        start_in(0, 0)

        def body(i, carry):
            p = i % 2
            wait_in(i, p)

            @pl.when(i >= 1)
            def _():
                wait_out(i - 1, 1 - p)

            @pl.when(i + 1 < nfull)
            def _():
                start_in(i + 1, 1 - p)

            start_out(i, p)
            return carry

        lax.fori_loop(0, nfull, body, 0)
        wait_out(nfull - 1, (nfull - 1) % 2)

    def tail(i, carry):
        r0 = nfull * C + i
        pltpu.sync_copy(src_hbm.at[r0, b], buf.at[0, 0])
        pltpu.sync_copy(buf.at[0, 0], dst_hbm.at[dst_off + r0, b])
        return carry

    lax.fori_loop(0, n_rows - nfull * C, tail, 0)


def _sc_body(L1, L2, seq1_hbm, seq2_hbm, len_hbm, out_hbm,
             lenv, zbuf, buf, isem, osem, zsem):
    b = lax.axis_index("s")
    role = lax.axis_index("c")
    H = zbuf.shape[1]

    pltpu.sync_copy(len_hbm.at[b], lenv)
    n1 = lenv[...][0]

    @pl.when(role == 0)
    def _():
        _copy_pipelined(seq1_hbm, out_hbm, b, n1, 0, buf, isem, osem)

        def zinit(j, carry):
            row = j // (H // 16)
            col = (j % (H // 16)) * 16
            zbuf[row, pl.ds(col, 16)] = jnp.zeros((16,), jnp.float32)
            return carry

        lax.fori_loop(0, _ZCHUNK * (H // 16), zinit, 0)

        zstart = n1 + L2
        nz = L1 - n1
        nzfull = nz // _ZCHUNK

        def zwait_one():
            pltpu.make_async_copy(
                zbuf, out_hbm.at[pl.ds(zstart, _ZCHUNK), b], zsem).wait()

        def zchunk(i, carry):
            @pl.when(i >= _ZLAG)
            def _():
                zwait_one()

            s0 = zstart + i * _ZCHUNK
            pltpu.async_copy(zbuf, out_hbm.at[pl.ds(s0, _ZCHUNK), b], zsem)
            return carry

        lax.fori_loop(0, nzfull, zchunk, 0)

        def zdrain(i, carry):
            zwait_one()
            return carry

        lax.fori_loop(0, jnp.minimum(nzfull, _ZLAG), zdrain, 0)

        def ztail(i, carry):
            r0 = zstart + nzfull * _ZCHUNK + i
            pltpu.sync_copy(zbuf.at[0], out_hbm.at[r0, b])
            return carry

        lax.fori_loop(0, nz - nzfull * _ZCHUNK, ztail, 0)

    @pl.when(role == 1)
    def _():
        _copy_pipelined(seq2_hbm, out_hbm, b, L2, n1, buf, isem, osem)


def kernel(seq1, mask1, seq2, mask2):
    L1, B, H = seq1.shape
    L2 = seq2.shape[0]
    LT = L1 + L2

    lens16, maski = pl.pallas_call(
        _meta_body,
        out_shape=(jax.ShapeDtypeStruct((B, 16), jnp.int32),
                   jax.ShapeDtypeStruct((B, LT), jnp.int32)),
    )(mask1.astype(jnp.int32), mask2.astype(jnp.int32))

    mesh = plsc.VectorSubcoreMesh(core_axis_name="c", subcore_axis_name="s")
    body = lambda *refs: _sc_body(L1, L2, *refs)
    out = pl.kernel(
        body,
        out_type=jax.ShapeDtypeStruct((LT, B, H), jnp.float32),
        mesh=mesh,
        scratch_types=[
            pltpu.VMEM((16,), jnp.int32),
            pltpu.VMEM((_ZCHUNK, H), jnp.float32),
            pltpu.VMEM((2, _CHUNK, H), jnp.float32),
            pltpu.SemaphoreType.DMA((2,)),
            pltpu.SemaphoreType.DMA((2,)),
            pltpu.SemaphoreType.DMA,
        ],
    )(seq1, seq2, lens16)

    return (out, maski.astype(bool))
